# Initial kernel scaffold; baseline (speedup 1.0000x reference)
import jax
import jax.numpy as jnp
from jax.experimental import pallas as pl


def kernel(hidden_states, W, b):
    raise NotImplementedError("write your pallas kernel here")


# speedup vs baseline: 20.7598x; 20.7598x over previous
import functools

import jax
import jax.numpy as jnp
from jax.experimental import pallas as pl

_K = 32
_SEQ = 2048
_GROUPS = _SEQ // _K


def _ce_split(x, j):
    g = _K // (2 * j)
    xr = x.reshape((g, 2, j) + x.shape[1:])
    return xr[:, 0], xr[:, 1], g


def _bitonic_sort(x, desc):
    k = 2
    while k <= _K:
        j = k // 2
        while j >= 1:
            a, b, g = _ce_split(x, j)
            mn = jnp.minimum(a, b)
            mx = jnp.maximum(a, b)
            parts = []
            for gi in range(g):
                desc_block = (((gi * 2 * j) & k) == 0) == desc
                if desc_block:
                    parts.append(mx[gi])
                    parts.append(mn[gi])
                else:
                    parts.append(mn[gi])
                    parts.append(mx[gi])
            x = jnp.concatenate(parts, axis=0)
            j //= 2
        k *= 2
    return x


def _bitonic_merge(x, desc):
    j = _K // 2
    while j >= 1:
        a, b, g = _ce_split(x, j)
        mn = jnp.minimum(a, b)
        mx = jnp.maximum(a, b)
        parts = []
        for gi in range(g):
            if desc:
                parts.append(mx[gi])
                parts.append(mn[gi])
            else:
                parts.append(mn[gi])
                parts.append(mx[gi])
        x = jnp.concatenate(parts, axis=0)
        j //= 2
    return x


def _topk_pool_body(x_ref, o_ref):
    x = x_ref[0]
    hb = x.shape[-1]
    x = x.reshape(_K, _GROUPS, hb)
    half = _GROUPS // 2
    x = jnp.concatenate(
        [_bitonic_sort(x[:, :half], True), _bitonic_sort(x[:, half:], False)],
        axis=1)
    ng = _GROUPS
    while ng > 2:
        h = ng // 2
        top = jnp.maximum(x[:, :h], x[:, h:])
        q = h // 2
        x = jnp.concatenate(
            [_bitonic_merge(top[:, :q], True), _bitonic_merge(top[:, q:], False)],
            axis=1)
        ng = h
    top = jnp.maximum(x[:, 0], x[:, 1])
    pooled = jnp.sum(top, axis=0) * (1.0 / _K)
    o_ref[0, 0, 0] = pooled


def _proj_body(p_ref, w_ref, b_ref, o_ref):
    acc = jax.lax.dot_general(
        p_ref[...], w_ref[...], (((1,), (1,)), ((), ())),
        preferred_element_type=jnp.float32)
    o_ref[...] = jnp.tanh(acc + b_ref[...])


@jax.jit
def kernel(hidden_states, W, b):
    bsz, seq, hid = hidden_states.shape
    hb = 512
    pooled = pl.pallas_call(
        _topk_pool_body,
        grid=(bsz, hid // hb),
        in_specs=[pl.BlockSpec((1, seq, hb), lambda i, j: (i, 0, j))],
        out_specs=pl.BlockSpec((1, 1, 1, hb), lambda i, j: (i, j, 0, 0)),
        out_shape=jax.ShapeDtypeStruct((bsz, hid // hb, 1, hb), jnp.float32),
    )(hidden_states)
    pooled = pooled.reshape(bsz, hid)

    ob = 512
    b2 = b.reshape(1, hid)
    out = pl.pallas_call(
        _proj_body,
        grid=(hid // ob,),
        in_specs=[
            pl.BlockSpec((bsz, hid), lambda o: (0, 0)),
            pl.BlockSpec((ob, hid), lambda o: (o, 0)),
            pl.BlockSpec((1, ob), lambda o: (0, o)),
        ],
        out_specs=pl.BlockSpec((bsz, ob), lambda o: (0, o)),
        out_shape=jax.ShapeDtypeStruct((bsz, hid), jnp.float32),
    )(pooled, W, b2)
    return out


# speedup vs baseline: 43.2802x; 2.0848x over previous
import jax
import jax.numpy as jnp
from jax.experimental import pallas as pl

_K = 32
_SEQ = 2048
_GROUPS = _SEQ // _K


def _sort32_list(v, desc):
    k = 2
    while k <= _K:
        j = k // 2
        while j >= 1:
            for i in range(_K):
                l = i ^ j
                if l > i:
                    a, b = v[i], v[l]
                    mx = jnp.maximum(a, b)
                    mn = jnp.minimum(a, b)
                    if (((i & k) == 0) == desc):
                        v[i], v[l] = mx, mn
                    else:
                        v[i], v[l] = mn, mx
            j //= 2
        k *= 2


def _cleanup_list(v, desc):
    j = _K // 2
    while j >= 1:
        for i in range(_K):
            l = i ^ j
            if l > i:
                a, b = v[i], v[l]
                mx = jnp.maximum(a, b)
                mn = jnp.minimum(a, b)
                if desc:
                    v[i], v[l] = mx, mn
                else:
                    v[i], v[l] = mn, mx
        j //= 2


def _cleanup_list_masked(v, dmask):
    j = _K // 2
    while j >= 1:
        for i in range(_K):
            l = i ^ j
            if l > i:
                a, b = v[i], v[l]
                mx = jnp.maximum(a, b)
                mn = jnp.minimum(a, b)
                v[i] = jnp.where(dmask, mx, mn)
                v[l] = jnp.where(dmask, mn, mx)
        j //= 2


def _merge_lists(va, vb, desc):
    t = [jnp.maximum(va[p], vb[p]) for p in range(_K)]
    _cleanup_list(t, desc)
    return t


def _ce_split(x, j):
    g = _K // (2 * j)
    xr = x.reshape((g, 2, j) + x.shape[1:])
    return xr[:, 0], xr[:, 1], g


def _bitonic_merge(x, desc):
    j = _K // 2
    while j >= 1:
        a, b, g = _ce_split(x, j)
        mn = jnp.minimum(a, b)
        mx = jnp.maximum(a, b)
        parts = []
        for gi in range(g):
            if desc:
                parts.append(mx[gi])
                parts.append(mn[gi])
            else:
                parts.append(mn[gi])
                parts.append(mx[gi])
        x = jnp.concatenate(parts, axis=0)
        j //= 2
    return x


def _topk_pool_body(x_ref, o_ref):
    x = x_ref[0]
    hb = x.shape[-1]
    x = x.reshape(_K, _GROUPS, hb)
    dmask = jax.lax.broadcasted_iota(jnp.int32, (8, 128), 0) < 4
    outs = []
    for c in range(hb // 128):
        def tile(p, gt):
            return x[p, gt * 8:(gt + 1) * 8, c * 128:(c + 1) * 128]

        def sorted_tile(gt, desc):
            v = [tile(p, gt) for p in range(_K)]
            _sort32_list(v, desc)
            return v

        o0 = _merge_lists(sorted_tile(0, True), sorted_tile(4, False), True)
        o2 = _merge_lists(sorted_tile(2, True), sorted_tile(6, False), False)
        q0 = _merge_lists(o0, o2, True)
        del o0, o2
        o1 = _merge_lists(sorted_tile(1, True), sorted_tile(5, False), True)
        o3 = _merge_lists(sorted_tile(3, True), sorted_tile(7, False), False)
        q1 = _merge_lists(o1, o3, False)
        del o1, o3
        r = [jnp.maximum(q0[p], q1[p]) for p in range(_K)]
        _cleanup_list_masked(r, dmask)
        xs = jnp.stack(r, axis=0)
        ng = 8
        while ng > 2:
            h = ng // 2
            top = jnp.maximum(xs[:, :h], xs[:, h:])
            q = h // 2
            xs = jnp.concatenate(
                [_bitonic_merge(top[:, :q], True),
                 _bitonic_merge(top[:, q:], False)], axis=1)
            ng = h
        t = jnp.maximum(xs[:, 0], xs[:, 1])
        outs.append(jnp.sum(t, axis=0) * (1.0 / _K))
    o_ref[0, 0, 0] = jnp.concatenate(outs)


def _proj_body(p_ref, w_ref, b_ref, o_ref):
    acc = jax.lax.dot_general(
        p_ref[...], w_ref[...], (((1,), (1,)), ((), ())),
        preferred_element_type=jnp.float32)
    o_ref[...] = jnp.tanh(acc + b_ref[...])


@jax.jit
def kernel(hidden_states, W, b):
    bsz, seq, hid = hidden_states.shape
    hb = 512
    pooled = pl.pallas_call(
        _topk_pool_body,
        grid=(bsz, hid // hb),
        in_specs=[pl.BlockSpec((1, seq, hb), lambda i, j: (i, 0, j))],
        out_specs=pl.BlockSpec((1, 1, 1, hb), lambda i, j: (i, j, 0, 0)),
        out_shape=jax.ShapeDtypeStruct((bsz, hid // hb, 1, hb), jnp.float32),
    )(hidden_states)
    pooled = pooled.reshape(bsz, hid)

    ob = 512
    b2 = b.reshape(1, hid)
    out = pl.pallas_call(
        _proj_body,
        grid=(hid // ob,),
        in_specs=[
            pl.BlockSpec((bsz, hid), lambda o: (0, 0)),
            pl.BlockSpec((ob, hid), lambda o: (o, 0)),
            pl.BlockSpec((1, ob), lambda o: (0, o)),
        ],
        out_specs=pl.BlockSpec((bsz, ob), lambda o: (0, o)),
        out_shape=jax.ShapeDtypeStruct((bsz, hid), jnp.float32),
    )(pooled, W, b2)
    return out


# speedup vs baseline: 60.3150x; 1.3936x over previous
import jax
import jax.numpy as jnp
from jax.experimental import pallas as pl

_K = 32
_SEQ = 2048
_GROUPS = _SEQ // _K


def _ce(v, i, l, desc):
    a, b = v[i], v[l]
    mx = jnp.maximum(a, b)
    mn = jnp.minimum(a, b)
    if desc:
        v[i], v[l] = mx, mn
    else:
        v[i], v[l] = mn, mx


def _oem_merge(v, lo, n, r, desc):
    m = r * 2
    if m < n:
        _oem_merge(v, lo, n, m, desc)
        _oem_merge(v, lo + r, n, m, desc)
        for i in range(lo + r, lo + n - r, m):
            _ce(v, i, i + r, desc)
    else:
        _ce(v, lo, lo + r, desc)


def _sort32_list(v, desc, lo=0, n=_K):
    if n > 1:
        m = n // 2
        _sort32_list(v, desc, lo, m)
        _sort32_list(v, desc, lo + m, m)
        _oem_merge(v, lo, n, 1, desc)


def _cleanup_list(v, desc):
    j = _K // 2
    while j >= 1:
        for i in range(_K):
            l = i ^ j
            if l > i:
                a, b = v[i], v[l]
                mx = jnp.maximum(a, b)
                mn = jnp.minimum(a, b)
                if desc:
                    v[i], v[l] = mx, mn
                else:
                    v[i], v[l] = mn, mx
        j //= 2


def _cleanup_list_masked(v, dmask):
    j = _K // 2
    while j >= 1:
        for i in range(_K):
            l = i ^ j
            if l > i:
                a, b = v[i], v[l]
                mx = jnp.maximum(a, b)
                mn = jnp.minimum(a, b)
                v[i] = jnp.where(dmask, mx, mn)
                v[l] = jnp.where(dmask, mn, mx)
        j //= 2


def _merge_lists(va, vb, desc):
    t = [jnp.maximum(va[p], vb[p]) for p in range(_K)]
    _cleanup_list(t, desc)
    return t


def _ce_split(x, j):
    g = _K // (2 * j)
    xr = x.reshape((g, 2, j) + x.shape[1:])
    return xr[:, 0], xr[:, 1], g


def _bitonic_merge(x, desc):
    j = _K // 2
    while j >= 1:
        a, b, g = _ce_split(x, j)
        mn = jnp.minimum(a, b)
        mx = jnp.maximum(a, b)
        parts = []
        for gi in range(g):
            if desc:
                parts.append(mx[gi])
                parts.append(mn[gi])
            else:
                parts.append(mn[gi])
                parts.append(mx[gi])
        x = jnp.concatenate(parts, axis=0)
        j //= 2
    return x


def _topk_pool_body(x_ref, o_ref):
    x = x_ref[0]
    hb = x.shape[-1]
    x = x.reshape(_K, _GROUPS, hb)
    dmask = jax.lax.broadcasted_iota(jnp.int32, (8, 128), 0) < 4
    pmask = jax.lax.broadcasted_iota(jnp.int32, (16, 128), 0) < 8
    outs = []
    for c in range(hb // 128):
        def ptile(p, gta, gtb):
            t = x[p, gta * 8:(gta + 1) * 8, c * 128:(c + 1) * 128]
            u = x[p, gtb * 8:(gtb + 1) * 8, c * 128:(c + 1) * 128]
            return jnp.concatenate([t, u], axis=0).astype(jnp.bfloat16)

        def sorted_pack(gta, gtb, desc):
            v = [ptile(p, gta, gtb) for p in range(_K)]
            _sort32_list(v, desc)
            return v

        q01 = _merge_lists(sorted_pack(0, 1, True), sorted_pack(4, 5, False),
                           True)
        q23 = _merge_lists(sorted_pack(2, 3, True), sorted_pack(6, 7, False),
                           False)
        r = [jnp.maximum(q01[p], q23[p]) for p in range(_K)]
        _cleanup_list_masked(r, pmask)
        r = [jnp.maximum(r[p][:8], r[p][8:]).astype(jnp.float32)
             for p in range(_K)]
        _cleanup_list_masked(r, dmask)
        xs = jnp.stack(r, axis=0)
        ng = 8
        while ng > 2:
            h = ng // 2
            top = jnp.maximum(xs[:, :h], xs[:, h:])
            q = h // 2
            xs = jnp.concatenate(
                [_bitonic_merge(top[:, :q], True),
                 _bitonic_merge(top[:, q:], False)], axis=1)
            ng = h
        t = jnp.maximum(xs[:, 0], xs[:, 1])
        outs.append(jnp.sum(t, axis=0) * (1.0 / _K))
    o_ref[0, 0, 0] = jnp.concatenate(outs)


def _proj_body(p_ref, w_ref, b_ref, o_ref):
    acc = jax.lax.dot_general(
        p_ref[...], w_ref[...], (((1,), (1,)), ((), ())),
        preferred_element_type=jnp.float32)
    o_ref[...] = jnp.tanh(acc + b_ref[...])


@jax.jit
def kernel(hidden_states, W, b):
    bsz, seq, hid = hidden_states.shape
    hb = 512
    pooled = pl.pallas_call(
        _topk_pool_body,
        grid=(bsz, hid // hb),
        in_specs=[pl.BlockSpec((1, seq, hb), lambda i, j: (i, 0, j))],
        out_specs=pl.BlockSpec((1, 1, 1, hb), lambda i, j: (i, j, 0, 0)),
        out_shape=jax.ShapeDtypeStruct((bsz, hid // hb, 1, hb), jnp.float32),
    )(hidden_states)
    pooled = pooled.reshape(bsz, hid)

    ob = 512
    b2 = b.reshape(1, hid)
    out = pl.pallas_call(
        _proj_body,
        grid=(hid // ob,),
        in_specs=[
            pl.BlockSpec((bsz, hid), lambda o: (0, 0)),
            pl.BlockSpec((ob, hid), lambda o: (o, 0)),
            pl.BlockSpec((1, ob), lambda o: (0, o)),
        ],
        out_specs=pl.BlockSpec((bsz, ob), lambda o: (0, o)),
        out_shape=jax.ShapeDtypeStruct((bsz, hid), jnp.float32),
    )(pooled, W, b2)
    return out


# speedup vs baseline: 63.4934x; 1.0527x over previous
import jax
import jax.numpy as jnp
from jax.experimental import pallas as pl
from jax.experimental.pallas import tpu as pltpu

_K = 32
_SEQ = 2048
_GROUPS = _SEQ // _K


def _ce(v, i, l, desc):
    a, b = v[i], v[l]
    mx = jnp.maximum(a, b)
    mn = jnp.minimum(a, b)
    if desc:
        v[i], v[l] = mx, mn
    else:
        v[i], v[l] = mn, mx


def _oem_merge(v, lo, n, r, desc):
    m = r * 2
    if m < n:
        _oem_merge(v, lo, n, m, desc)
        _oem_merge(v, lo + r, n, m, desc)
        for i in range(lo + r, lo + n - r, m):
            _ce(v, i, i + r, desc)
    else:
        _ce(v, lo, lo + r, desc)


def _sort32_list(v, desc, lo=0, n=_K):
    if n > 1:
        m = n // 2
        _sort32_list(v, desc, lo, m)
        _sort32_list(v, desc, lo + m, m)
        _oem_merge(v, lo, n, 1, desc)


def _cleanup_list(v, desc):
    j = _K // 2
    while j >= 1:
        for i in range(_K):
            l = i ^ j
            if l > i:
                a, b = v[i], v[l]
                mx = jnp.maximum(a, b)
                mn = jnp.minimum(a, b)
                if desc:
                    v[i], v[l] = mx, mn
                else:
                    v[i], v[l] = mn, mx
        j //= 2


def _cleanup_list_masked(v, dmask):
    j = _K // 2
    while j >= 1:
        for i in range(_K):
            l = i ^ j
            if l > i:
                a, b = v[i], v[l]
                mx = jnp.maximum(a, b)
                mn = jnp.minimum(a, b)
                v[i] = jnp.where(dmask, mx, mn)
                v[l] = jnp.where(dmask, mn, mx)
        j //= 2


def _merge_lists(va, vb, desc):
    t = [jnp.maximum(va[p], vb[p]) for p in range(_K)]
    _cleanup_list(t, desc)
    return t


def _ce_split(x, j):
    g = _K // (2 * j)
    xr = x.reshape((g, 2, j) + x.shape[1:])
    return xr[:, 0], xr[:, 1], g


def _bitonic_merge(x, desc):
    j = _K // 2
    while j >= 1:
        a, b, g = _ce_split(x, j)
        mn = jnp.minimum(a, b)
        mx = jnp.maximum(a, b)
        parts = []
        for gi in range(g):
            if desc:
                parts.append(mx[gi])
                parts.append(mn[gi])
            else:
                parts.append(mn[gi])
                parts.append(mx[gi])
        x = jnp.concatenate(parts, axis=0)
        j //= 2
    return x


def _topk_pool(x):
    hb = x.shape[-1]
    x = x.reshape(_K, _GROUPS, hb)
    dmask = jax.lax.broadcasted_iota(jnp.int32, (8, 128), 0) < 4
    pmask = jax.lax.broadcasted_iota(jnp.int32, (16, 128), 0) < 8
    outs = []
    for c in range(hb // 128):
        def ptile(p, gta, gtb):
            t = x[p, gta * 8:(gta + 1) * 8, c * 128:(c + 1) * 128]
            u = x[p, gtb * 8:(gtb + 1) * 8, c * 128:(c + 1) * 128]
            return jnp.concatenate([t, u], axis=0).astype(jnp.bfloat16)

        def sorted_pack(gta, gtb, desc):
            v = [ptile(p, gta, gtb) for p in range(_K)]
            _sort32_list(v, desc)
            return v

        q01 = _merge_lists(sorted_pack(0, 1, True), sorted_pack(4, 5, False),
                           True)
        q23 = _merge_lists(sorted_pack(2, 3, True), sorted_pack(6, 7, False),
                           False)
        r = [jnp.maximum(q01[p], q23[p]) for p in range(_K)]
        _cleanup_list_masked(r, pmask)
        r = [jnp.maximum(r[p][:8], r[p][8:]).astype(jnp.float32)
             for p in range(_K)]
        _cleanup_list_masked(r, dmask)
        xs = jnp.stack(r, axis=0)
        ng = 8
        while ng > 2:
            h = ng // 2
            top = jnp.maximum(xs[:, :h], xs[:, h:])
            q = h // 2
            xs = jnp.concatenate(
                [_bitonic_merge(top[:, :q], True),
                 _bitonic_merge(top[:, q:], False)], axis=1)
            ng = h
        t = jnp.maximum(xs[:, 0], xs[:, 1])
        outs.append(jnp.sum(t, axis=0) * (1.0 / _K))
    return jnp.concatenate(outs)


def _fused_body(x_ref, w_ref, b_ref, o_ref, pooled_ref):
    j = pl.program_id(0)
    bi = pl.program_id(1)
    nj = pl.num_programs(0)
    pooled = _topk_pool(x_ref[0])
    pooled_ref[pl.ds(bi, 1), :] = pooled[None, :]

    @pl.when(bi == pl.num_programs(1) - 1)
    def _():
        partial = jax.lax.dot_general(
            pooled_ref[...], w_ref[...], (((1,), (1,)), ((), ())),
            preferred_element_type=jnp.float32)

        @pl.when(j == 0)
        def _():
            o_ref[...] = partial + b_ref[...]

        @pl.when(jnp.logical_and(j > 0, j < nj - 1))
        def _():
            o_ref[...] = o_ref[...] + partial

        @pl.when(j == nj - 1)
        def _():
            o_ref[...] = jnp.tanh(o_ref[...] + partial)


@jax.jit
def kernel(hidden_states, W, b):
    bsz, seq, hid = hidden_states.shape
    hb = 512
    b2 = b.reshape(1, hid)
    out = pl.pallas_call(
        _fused_body,
        grid=(hid // hb, bsz),
        in_specs=[
            pl.BlockSpec((1, seq, hb), lambda j, i: (i, 0, j)),
            pl.BlockSpec((hid, hb), lambda j, i: (0, j)),
            pl.BlockSpec((1, hid), lambda j, i: (0, 0)),
        ],
        out_specs=pl.BlockSpec((bsz, hid), lambda j, i: (0, 0)),
        out_shape=jax.ShapeDtypeStruct((bsz, hid), jnp.float32),
        scratch_shapes=[pltpu.VMEM((4, hb), jnp.float32)],
    )(hidden_states, W, b2)
    return out
